# baseline (speedup 1.0000x reference)
import jax
import jax.numpy as jnp
from jax.experimental import pallas as pl
from jax.experimental.pallas import tpu as pltpu


def _add_body(x_ref, pos_ref, o_ref):
    s = o_ref.shape[1]
    o_ref[...] = x_ref[...] + pos_ref[:s][None, :, :]


def kernel(x, pos_table):
    B, S, D = x.shape
    return pl.pallas_call(
        _add_body,
        grid=(B,),
        in_specs=[
            pl.BlockSpec((1, S, D), lambda i: (i, 0, 0)),
            pl.BlockSpec(pos_table.shape, lambda i: (0, 0)),
        ],
        out_specs=pl.BlockSpec((1, S, D), lambda i: (i, 0, 0)),
        out_shape=jax.ShapeDtypeStruct((B, S, D), x.dtype),
        compiler_params=pltpu.CompilerParams(
            dimension_semantics=("parallel",),
        ),
    )(x, pos_table)


# speedup vs baseline: 1.0471x; 1.0471x over previous
import jax
import jax.numpy as jnp
from jax.experimental import pallas as pl
from jax.experimental.pallas import tpu as pltpu


def _add_body(x_ref, pos_ref, o_ref):
    s = o_ref.shape[1]
    o_ref[...] = x_ref[...] + pos_ref[:s][None, :, :]


def kernel(x, pos_table):
    B, S, D = x.shape
    BBLK = 4
    return pl.pallas_call(
        _add_body,
        grid=(B // BBLK,),
        in_specs=[
            pl.BlockSpec((BBLK, S, D), lambda i: (i, 0, 0)),
            pl.BlockSpec(pos_table.shape, lambda i: (0, 0)),
        ],
        out_specs=pl.BlockSpec((BBLK, S, D), lambda i: (i, 0, 0)),
        out_shape=jax.ShapeDtypeStruct((B, S, D), x.dtype),
        compiler_params=pltpu.CompilerParams(
            dimension_semantics=("parallel",),
        ),
    )(x, pos_table)
